# baseline (speedup 1.0000x reference)
import functools

import jax
import jax.numpy as jnp
from jax import lax
from jax.experimental import pallas as pl
from jax.experimental.pallas import tpu as pltpu
from jax.experimental.pallas import tpu_sc as plsc

_N = 10000
_E = 160000
_EMB = 128
_HALF = 64
_G = 128
_L = 4
_EPS = 1e-7

_NC = 2
_NS = 16
_C = 40
_EPT = _E // _NS
_NCHUNK = _EPT // _C
_NP = 10240
_RPT = _NP // _NS
_ZR = _C

_F32 = jnp.float32


def _sc_body(hfull, eelo, eehi, eidx4, mlo, mhi, tvec, olo, ohi,
             acc, gb0, gb1, eb0, eb1, pp0, pp1, ib0, ib1, db0, db1, mv, tb,
             sg0, sg1, se0, se1, ss0, ss1, si0, si1, sd0, sd1, sw0, sw1):
    c = lax.axis_index("c")
    s = lax.axis_index("s")
    gb = (gb0, gb1)
    eb = (eb0, eb1)
    pp = (pp0, pp1)
    ib = (ib0, ib1)
    db = (db0, db1)
    si = (si0, si1)
    sd = (sd0, sd1)
    zb = (pp0, pp1)
    sg = (sg0, sg1)
    se = (se0, se1)
    ss = (ss0, ss1)
    sw = (sw0, sw1)

    @pl.when(c == 0)
    def _():
        pltpu.sync_copy(mlo, mv)

    @pl.when(c == 1)
    def _():
        pltpu.sync_copy(mhi, mv)

    pltpu.sync_copy(tvec, tb)
    tv = tb[...]
    mj = [mv[pl.ds(16 * j, 16)] for j in range(4)]

    @pl.loop(0, _ZR)
    def _zbl(i):
        for j in range(8):
            pp0[i, pl.ds(16 * j, 16)] = jnp.zeros((16,), _F32)

    row0 = s * _RPT
    nzc = _RPT // _ZR

    for z in range(nzc):
        rows = pl.ds(pl.multiple_of(row0 + z * _ZR, 8), _ZR)
        pltpu.async_copy(pp0, acc.at[rows], sw0)
    for z in range(nzc):
        rows = pl.ds(pl.multiple_of(row0 + z * _ZR, 8), _ZR)
        pltpu.make_async_copy(pp0, acc.at[rows], sw0).wait()

    plsc.subcore_barrier()

    ebase = s * _EPT

    def _issue_src(k, sl):
        pltpu.async_copy(eidx4.at[0, s, k], ib[sl], si[sl])

    def _wait_src(sl):
        pltpu.make_async_copy(eidx4.at[0, s, 0], ib[sl], si[sl]).wait()

    def _issue_dst(k, sl):
        pltpu.async_copy(eidx4.at[1, s, k], db[sl], sd[sl])

    def _wait_dst(sl):
        pltpu.make_async_copy(eidx4.at[1, s, 0], db[sl], sd[sl]).wait()

    def _issue(k, sl):
        base = pl.multiple_of(ebase + k * _C, 8)
        pltpu.async_copy(hfull.at[ib[sl]], gb[sl], sg[sl])

        @pl.when(c == 0)
        def _():
            pltpu.async_copy(eelo.at[pl.ds(base, _C)], eb[sl], se[sl])

        @pl.when(c == 1)
        def _():
            pltpu.async_copy(eehi.at[pl.ds(base, _C)], eb[sl], se[sl])

    def _wait_in(sl):
        pltpu.make_async_copy(hfull.at[ib[sl]], gb[sl], sg[sl]).wait()
        pltpu.make_async_copy(eelo.at[pl.ds(0, _C)], eb[sl], se[sl]).wait()

    def _issue_sc(sl):
        pltpu.async_copy(pp[sl], acc.at[db[sl]], ss[sl], add=True)

    def _wait_sc(sl):
        pltpu.make_async_copy(pp[sl], acc.at[db[sl]], ss[sl]).wait()

    def _compute(k, sl):
        gbuf = gb[sl]
        eebuf = eb[sl]
        pmp = pp[sl]

        def _body(goff):
            @pl.loop(0, _C, unroll=4)
            def _row(r):
                for j in range(4):
                    g = gbuf[r, pl.ds(goff + 16 * j, 16)]
                    e = eebuf[r, pl.ds(16 * j, 16)]
                    rr = jnp.maximum(g + e, 0.0)
                    p = jnp.exp(rr * tv - mj[j])
                    pmp[r, pl.ds(16 * j, 16)] = p * rr
                    pmp[r, pl.ds(_HALF + 16 * j, 16)] = p

        @pl.when(c == 0)
        def _():
            _body(0)

        @pl.when(c == 1)
        def _():
            _body(_HALF)

    _issue_src(0, 0)
    _issue_src(1, 1)
    _wait_src(0)
    _issue(0, 0)
    _wait_src(1)
    _issue(1, 1)

    npair = _NCHUNK // 2

    @pl.loop(0, npair)
    def _pair(m):
        a = 2 * m
        for par in range(2):
            k = a + par
            _wait_in(par)

            @pl.when(m > 0)
            def _():
                _wait_sc(par)

            _issue_dst(k, par)

            @pl.when(k + 2 < _NCHUNK)
            def _():
                _issue_src(k + 2, par)

            _compute(k, par)
            _wait_dst(par)
            _issue_sc(par)

            @pl.when(k + 2 < _NCHUNK)
            def _():
                _wait_src(par)
                _issue(k + 2, par)

    _wait_sc(0)
    _wait_sc(1)

    plsc.subcore_barrier()

    def _wo_one(z, par):
        rows = pl.ds(pl.multiple_of(row0 + z * _ZR, 8), _ZR)
        pltpu.sync_copy(acc.at[rows], zb[par])

        @pl.when(c == 0)
        def _():
            pltpu.async_copy(zb[par], olo.at[rows], sw[par])

        @pl.when(c == 1)
        def _():
            pltpu.async_copy(zb[par], ohi.at[rows], sw[par])

    def _wo_wait(par):
        pltpu.make_async_copy(zb[par], olo.at[pl.ds(0, _ZR)], sw[par]).wait()

    @pl.loop(0, nzc // 2)
    def _wo(z):
        for par in range(2):
            @pl.when(z > 0)
            def _():
                _wo_wait(par)

            _wo_one(2 * z + par, par)

    _wo_wait(0)
    _wo_wait(1)


_SC_KERNEL_CACHE = []


def _sc_aggregate(*args):
    if not _SC_KERNEL_CACHE:
        _SC_KERNEL_CACHE.append(_build_sc_kernel())
    return _SC_KERNEL_CACHE[0](*args)


def _build_sc_kernel():
    return pl.kernel(
        _sc_body,
        out_type=(
            jax.ShapeDtypeStruct((_NP, 2 * _HALF), _F32),
            jax.ShapeDtypeStruct((_NP, 2 * _HALF), _F32),
        ),
        mesh=plsc.VectorSubcoreMesh(
            core_axis_name="c", subcore_axis_name="s",
            num_cores=_NC, num_subcores=_NS),
        scratch_types=[
        pltpu.VMEM_SHARED((_NP, 2 * _HALF), _F32),
        pltpu.VMEM((_C, 2 * _HALF), _F32),
        pltpu.VMEM((_C, 2 * _HALF), _F32),
        pltpu.VMEM((_C, _HALF), _F32),
        pltpu.VMEM((_C, _HALF), _F32),
        pltpu.VMEM((_C, 2 * _HALF), _F32),
        pltpu.VMEM((_C, 2 * _HALF), _F32),
        pltpu.VMEM((_C,), jnp.int32),
        pltpu.VMEM((_C,), jnp.int32),
        pltpu.VMEM((_C,), jnp.int32),
        pltpu.VMEM((_C,), jnp.int32),
        pltpu.VMEM((_HALF,), _F32),
        pltpu.VMEM((16,), _F32),
        pltpu.SemaphoreType.DMA,
        pltpu.SemaphoreType.DMA,
        pltpu.SemaphoreType.DMA,
        pltpu.SemaphoreType.DMA,
        pltpu.SemaphoreType.DMA,
        pltpu.SemaphoreType.DMA,
        pltpu.SemaphoreType.DMA,
        pltpu.SemaphoreType.DMA,
        pltpu.SemaphoreType.DMA,
        pltpu.SemaphoreType.DMA,
        pltpu.SemaphoreType.DMA,
        pltpu.SemaphoreType.DMA,
        ],
    )


def _edge_enc_body(ea_ref, w_ref, b_ref, eelo_ref, eehi_ref, emax_ref, accm):
    i = pl.program_id(0)
    ee = jnp.dot(ea_ref[...], w_ref[...], preferred_element_type=_F32)
    ee = ee + b_ref[...]
    eelo_ref[...] = ee[:, :_HALF]
    eehi_ref[...] = ee[:, _HALF:]
    bm = jnp.max(ee, axis=0, keepdims=True)

    @pl.when(i == 0)
    def _():
        accm[...] = bm

    @pl.when(i > 0)
    def _():
        accm[...] = jnp.maximum(accm[...], bm)

    emax_ref[...] = accm[...]


_TE = 4000


def _edge_encode(edge_attr, edge_W, edge_b):
    nblk = _E // _TE
    return pl.pallas_call(
        _edge_enc_body,
        grid=(nblk,),
        in_specs=[
            pl.BlockSpec((_TE, 16), lambda i: (i, 0)),
            pl.BlockSpec((16, _EMB), lambda i: (0, 0)),
            pl.BlockSpec((1, _EMB), lambda i: (0, 0)),
        ],
        out_specs=[
            pl.BlockSpec((_TE, _HALF), lambda i: (i, 0)),
            pl.BlockSpec((_TE, _HALF), lambda i: (i, 0)),
            pl.BlockSpec((1, _EMB), lambda i: (0, 0)),
        ],
        out_shape=[
            jax.ShapeDtypeStruct((_E, _HALF), _F32),
            jax.ShapeDtypeStruct((_E, _HALF), _F32),
            jax.ShapeDtypeStruct((1, _EMB), _F32),
        ],
        scratch_shapes=[pltpu.VMEM((1, _EMB), _F32)],
    )(edge_attr, edge_W, edge_b.reshape(1, _EMB))


def _node_enc_body(x_ref, w_ref, b_ref, emax_ref, t0_ref, h_ref, m_ref):
    h = jnp.dot(x_ref[...], w_ref[...], preferred_element_type=_F32)
    h = h + b_ref[...]
    h_ref[...] = h
    hmax = jnp.max(h, axis=0, keepdims=True)
    m_ref[...] = t0_ref[...] * jnp.maximum(hmax + emax_ref[...], 0.0)


def _node_encode(x, enc_W, enc_b, eemax, t0_row):
    return pl.pallas_call(
        _node_enc_body,
        out_shape=[
            jax.ShapeDtypeStruct((_N, _EMB), _F32),
            jax.ShapeDtypeStruct((1, _EMB), _F32),
        ],
    )(x, enc_W, enc_b.reshape(1, _EMB), eemax, t0_row)


def _aggr_from(o, hin):
    pm = o[:, :_HALF]
    den = o[:, _HALF:]
    return jnp.where(den > 0.0, pm / den + _EPS, 0.0) + hin


def _bn_tc(y, g, b):
    mu = jnp.mean(y, axis=0, keepdims=True)
    yc = y - mu
    var = jnp.mean(yc * yc, axis=0, keepdims=True)
    return yc * (g / jnp.sqrt(var + 1e-5)) + b


def _layer_body(has_resid, is_final, *refs):
    if has_resid:
        (olo, ohi, hin, hprev, w1, b1, g1, be1, w2, b2,
         ng, nb, emax, tnext, *outs) = refs
    else:
        (olo, ohi, hin, w1, b1, g1, be1, w2, b2,
         ng, nb, emax, tnext, *outs) = refs
        hprev = None

    hin_v = hin[...]
    out_lo = _aggr_from(olo[...][:_N], hin_v[:, :_HALF])
    out_hi = _aggr_from(ohi[...][:_N], hin_v[:, _HALF:])
    out = jnp.concatenate([out_lo, out_hi], axis=1)
    y = jnp.dot(out, w1[...], preferred_element_type=_F32) + b1[...]
    y = _bn_tc(y, g1[...], be1[...])
    y = jnp.maximum(y, 0.0)
    hnew = jnp.dot(y, w2[...], preferred_element_type=_F32) + b2[...]
    if hprev is not None:
        hnew = hnew + hprev[...]

    hb = _bn_tc(hnew, ng[...], nb[...])
    hin2 = jnp.maximum(hb, 0.0)
    if is_final:
        (hf_ref,) = outs
        hf_ref[...] = hin2
    else:
        (hnew_ref, hin2_ref, m_ref) = outs
        hnew_ref[...] = hnew
        hin2_ref[...] = hin2
        hmax = jnp.max(hin2, axis=0, keepdims=True)
        m_ref[...] = tnext[...] * jnp.maximum(hmax + emax[...], 0.0)


def _layer_tc(olo, ohi, hin, hprev, w1, b1, g1, be1, w2, b2,
              ng, nb, eemax, tnext_row, is_final):
    has_resid = hprev is not None
    if is_final:
        out_shape = [jax.ShapeDtypeStruct((_N, _EMB), _F32)]
    else:
        out_shape = [
            jax.ShapeDtypeStruct((_N, _EMB), _F32),
            jax.ShapeDtypeStruct((_N, _EMB), _F32),
            jax.ShapeDtypeStruct((1, _EMB), _F32),
        ]
    args = [olo, ohi, hin]
    if has_resid:
        args.append(hprev)
    args += [w1, b1.reshape(1, -1), g1.reshape(1, -1), be1.reshape(1, -1),
             w2, b2.reshape(1, -1), ng.reshape(1, -1), nb.reshape(1, -1),
             eemax, tnext_row]
    body = functools.partial(_layer_body, has_resid, is_final)
    return pl.pallas_call(body, out_shape=out_shape)(*args)


def _pool_body(hf_ref, batch_ref, pw_ref, pb_ref, rf_ref, beta_ref, out_ref):
    onehot = (lax.broadcasted_iota(jnp.int32, (_G, _N), 0)
              == batch_ref[...]).astype(_F32)
    sums = jnp.dot(onehot, hf_ref[...], preferred_element_type=_F32)
    counts = jnp.sum(onehot, axis=1, keepdims=True)
    hg = sums / jnp.maximum(counts, 1.0)
    pred = jnp.dot(hg, pw_ref[...], preferred_element_type=_F32) + pb_ref[...]
    sig = 1.0 / (1.0 + jnp.exp(-pred))
    out_ref[...] = (1.0 - beta_ref[...]) * sig + beta_ref[...] * rf_ref[...]


def _pool(hf, batch, pred_W, pred_b, rf_pred, beta):
    pw = jnp.zeros((_EMB, _EMB), _F32).at[:, 0:1].set(pred_W)
    pb = jnp.zeros((1, _EMB), _F32).at[0, 0].set(pred_b[0])
    rf = jnp.broadcast_to(rf_pred.reshape(_G, 1), (_G, _EMB))
    beta_row = jnp.broadcast_to(beta.reshape(1, 1), (1, _EMB))
    res = pl.pallas_call(
        _pool_body,
        out_shape=jax.ShapeDtypeStruct((_G, _EMB), _F32),
    )(hf, batch.reshape(1, _N), pw, pb, rf, beta_row)
    return res[:, 0:1]


def kernel(x, edge_index, edge_attr, batch, rf_pred, enc_W, enc_b, edge_W,
           edge_b, W1, b1, g1, be1, W2, b2, t_param, norm_g, norm_b, pred_W,
           pred_b, beta):
    eidx4 = edge_index.reshape(2, _NS, _NCHUNK, _C)

    eelo, eehi, eemax = _edge_encode(edge_attr, edge_W, edge_b)

    t_rows = [jnp.broadcast_to(t_param[l].reshape(1, 1), (1, _EMB))
              for l in range(_L)]
    t_vecs = [jnp.broadcast_to(t_param[l].reshape(1), (16,)) for l in range(_L)]

    hin, m_cur = _node_encode(x, enc_W, enc_b, eemax, t_rows[0])

    hprev = None
    hf = None
    for l in range(_L):
        olo, ohi = _sc_aggregate(hin, eelo, eehi, eidx4,
                                 m_cur[0, :_HALF], m_cur[0, _HALF:],
                                 t_vecs[l])
        is_final = l == _L - 1
        tnext = t_rows[l + 1] if not is_final else t_rows[l]
        outs = _layer_tc(olo, ohi, hin, hprev, W1[l], b1[l], g1[l],
                         be1[l], W2[l], b2[l], norm_g[l], norm_b[l], eemax,
                         tnext, is_final)
        if is_final:
            (hf,) = outs
        else:
            hprev, hin, m_cur = outs

    return _pool(hf, batch, pred_W, pred_b, rf_pred, beta)


# speedup vs baseline: 2.4527x; 2.4527x over previous
import functools

import jax
import jax.numpy as jnp
from jax import lax
from jax.experimental import pallas as pl
from jax.experimental.pallas import tpu as pltpu
from jax.experimental.pallas import tpu_sc as plsc

_N = 10000
_E = 160000
_EMB = 128
_HALF = 64
_G = 128
_L = 4
_EPS = 1e-7

_NC = 2
_NS = 16
_C = 40
_EPT = _E // _NS
_NCHUNK = _EPT // _C
_NP = 10240
_RPT = _NP // _NS
_ZR = _C

_F32 = jnp.float32


def _sc_body(hfull, eelo, eehi, eidx4, mlo, mhi, tvec, olo, ohi,
             acc, gb0, gb1, eb0, eb1, pp0, pp1, ib0, ib1, db0, db1, mv, tb,
             sg0, sg1, se0, se1, ss0, ss1, si0, si1, sd0, sd1, sw0, sw1):
    c = lax.axis_index("c")
    s = lax.axis_index("s")
    gb = (gb0, gb1)
    eb = (eb0, eb1)
    pp = (pp0, pp1)
    ib = (ib0, ib1)
    db = (db0, db1)
    si = (si0, si1)
    sd = (sd0, sd1)
    zb = (pp0, pp1)
    sg = (sg0, sg1)
    se = (se0, se1)
    ss = (ss0, ss1)
    sw = (sw0, sw1)

    @pl.when(c == 0)
    def _():
        pltpu.sync_copy(mlo, mv)

    @pl.when(c == 1)
    def _():
        pltpu.sync_copy(mhi, mv)

    pltpu.sync_copy(tvec, tb)
    tv = tb[...]
    mj = [mv[pl.ds(16 * j, 16)] for j in range(4)]

    @pl.loop(0, _ZR)
    def _zbl(i):
        for j in range(8):
            pp0[i, pl.ds(16 * j, 16)] = jnp.zeros((16,), _F32)

    row0 = s * _RPT
    nzc = _RPT // _ZR

    for z in range(nzc):
        rows = pl.ds(pl.multiple_of(row0 + z * _ZR, 8), _ZR)
        pltpu.async_copy(pp0, acc.at[rows], sw0)
    for z in range(nzc):
        rows = pl.ds(pl.multiple_of(row0 + z * _ZR, 8), _ZR)
        pltpu.make_async_copy(pp0, acc.at[rows], sw0).wait()

    plsc.subcore_barrier()

    ebase = s * _EPT

    def _issue_src(k, sl):
        pltpu.async_copy(eidx4.at[0, s, k], ib[sl], si[sl])

    def _wait_src(sl):
        pltpu.make_async_copy(eidx4.at[0, s, 0], ib[sl], si[sl]).wait()

    def _issue_dst(k, sl):
        pltpu.async_copy(eidx4.at[1, s, k], db[sl], sd[sl])

    def _wait_dst(sl):
        pltpu.make_async_copy(eidx4.at[1, s, 0], db[sl], sd[sl]).wait()

    def _issue(k, sl):
        base = pl.multiple_of(ebase + k * _C, 8)
        pltpu.async_copy(hfull.at[ib[sl]], gb[sl], sg[sl])

        @pl.when(c == 0)
        def _():
            pltpu.async_copy(eelo.at[pl.ds(base, _C)], eb[sl], se[sl])

        @pl.when(c == 1)
        def _():
            pltpu.async_copy(eehi.at[pl.ds(base, _C)], eb[sl], se[sl])

    def _wait_in(sl):
        pltpu.make_async_copy(hfull.at[ib[sl]], gb[sl], sg[sl]).wait()
        pltpu.make_async_copy(eelo.at[pl.ds(0, _C)], eb[sl], se[sl]).wait()

    def _issue_sc(sl):
        pltpu.async_copy(pp[sl], acc.at[db[sl]], ss[sl], add=True)

    def _wait_sc(sl):
        pltpu.make_async_copy(pp[sl], acc.at[db[sl]], ss[sl]).wait()

    def _compute(k, sl):
        gbuf = gb[sl]
        eebuf = eb[sl]
        pmp = pp[sl]

        def _body(goff):
            @plsc.parallel_loop(0, _C, unroll=8)
            def _row(r):
                for j in range(4):
                    g = gbuf[r, pl.ds(goff + 16 * j, 16)]
                    e = eebuf[r, pl.ds(16 * j, 16)]
                    rr = jnp.maximum(g + e, 0.0)
                    p = jnp.exp(rr * tv - mj[j])
                    pmp[r, pl.ds(16 * j, 16)] = p * rr
                    pmp[r, pl.ds(_HALF + 16 * j, 16)] = p

        @pl.when(c == 0)
        def _():
            _body(0)

        @pl.when(c == 1)
        def _():
            _body(_HALF)

    _issue_src(0, 0)
    _issue_src(1, 1)
    _wait_src(0)
    _issue(0, 0)
    _wait_src(1)
    _issue(1, 1)

    npair = _NCHUNK // 2

    @pl.loop(0, npair)
    def _pair(m):
        a = 2 * m
        for par in range(2):
            k = a + par
            _wait_in(par)

            @pl.when(m > 0)
            def _():
                _wait_sc(par)

            _issue_dst(k, par)

            @pl.when(k + 2 < _NCHUNK)
            def _():
                _issue_src(k + 2, par)

            _compute(k, par)
            _wait_dst(par)
            _issue_sc(par)

            @pl.when(k + 2 < _NCHUNK)
            def _():
                _wait_src(par)
                _issue(k + 2, par)

    _wait_sc(0)
    _wait_sc(1)

    plsc.subcore_barrier()

    def _wo_one(z, par):
        rows = pl.ds(pl.multiple_of(row0 + z * _ZR, 8), _ZR)
        pltpu.sync_copy(acc.at[rows], zb[par])

        @pl.when(c == 0)
        def _():
            pltpu.async_copy(zb[par], olo.at[rows], sw[par])

        @pl.when(c == 1)
        def _():
            pltpu.async_copy(zb[par], ohi.at[rows], sw[par])

    def _wo_wait(par):
        pltpu.make_async_copy(zb[par], olo.at[pl.ds(0, _ZR)], sw[par]).wait()

    @pl.loop(0, nzc // 2)
    def _wo(z):
        for par in range(2):
            @pl.when(z > 0)
            def _():
                _wo_wait(par)

            _wo_one(2 * z + par, par)

    _wo_wait(0)
    _wo_wait(1)


_SC_KERNEL_CACHE = []


def _sc_aggregate(*args):
    if not _SC_KERNEL_CACHE:
        _SC_KERNEL_CACHE.append(_build_sc_kernel())
    return _SC_KERNEL_CACHE[0](*args)


def _build_sc_kernel():
    return pl.kernel(
        _sc_body,
        out_type=(
            jax.ShapeDtypeStruct((_NP, 2 * _HALF), _F32),
            jax.ShapeDtypeStruct((_NP, 2 * _HALF), _F32),
        ),
        mesh=plsc.VectorSubcoreMesh(
            core_axis_name="c", subcore_axis_name="s",
            num_cores=_NC, num_subcores=_NS),
        scratch_types=[
        pltpu.VMEM_SHARED((_NP, 2 * _HALF), _F32),
        pltpu.VMEM((_C, 2 * _HALF), _F32),
        pltpu.VMEM((_C, 2 * _HALF), _F32),
        pltpu.VMEM((_C, _HALF), _F32),
        pltpu.VMEM((_C, _HALF), _F32),
        pltpu.VMEM((_C, 2 * _HALF), _F32),
        pltpu.VMEM((_C, 2 * _HALF), _F32),
        pltpu.VMEM((_C,), jnp.int32),
        pltpu.VMEM((_C,), jnp.int32),
        pltpu.VMEM((_C,), jnp.int32),
        pltpu.VMEM((_C,), jnp.int32),
        pltpu.VMEM((_HALF,), _F32),
        pltpu.VMEM((16,), _F32),
        pltpu.SemaphoreType.DMA,
        pltpu.SemaphoreType.DMA,
        pltpu.SemaphoreType.DMA,
        pltpu.SemaphoreType.DMA,
        pltpu.SemaphoreType.DMA,
        pltpu.SemaphoreType.DMA,
        pltpu.SemaphoreType.DMA,
        pltpu.SemaphoreType.DMA,
        pltpu.SemaphoreType.DMA,
        pltpu.SemaphoreType.DMA,
        pltpu.SemaphoreType.DMA,
        pltpu.SemaphoreType.DMA,
        ],
    )


def _edge_enc_body(ea_ref, w_ref, b_ref, eelo_ref, eehi_ref, emax_ref, accm):
    i = pl.program_id(0)
    ee = jnp.dot(ea_ref[...], w_ref[...], preferred_element_type=_F32)
    ee = ee + b_ref[...]
    eelo_ref[...] = ee[:, :_HALF]
    eehi_ref[...] = ee[:, _HALF:]
    bm = jnp.max(ee, axis=0, keepdims=True)

    @pl.when(i == 0)
    def _():
        accm[...] = bm

    @pl.when(i > 0)
    def _():
        accm[...] = jnp.maximum(accm[...], bm)

    emax_ref[...] = accm[...]


_TE = 4000


def _edge_encode(edge_attr, edge_W, edge_b):
    nblk = _E // _TE
    return pl.pallas_call(
        _edge_enc_body,
        grid=(nblk,),
        in_specs=[
            pl.BlockSpec((_TE, 16), lambda i: (i, 0)),
            pl.BlockSpec((16, _EMB), lambda i: (0, 0)),
            pl.BlockSpec((1, _EMB), lambda i: (0, 0)),
        ],
        out_specs=[
            pl.BlockSpec((_TE, _HALF), lambda i: (i, 0)),
            pl.BlockSpec((_TE, _HALF), lambda i: (i, 0)),
            pl.BlockSpec((1, _EMB), lambda i: (0, 0)),
        ],
        out_shape=[
            jax.ShapeDtypeStruct((_E, _HALF), _F32),
            jax.ShapeDtypeStruct((_E, _HALF), _F32),
            jax.ShapeDtypeStruct((1, _EMB), _F32),
        ],
        scratch_shapes=[pltpu.VMEM((1, _EMB), _F32)],
    )(edge_attr, edge_W, edge_b.reshape(1, _EMB))


def _node_enc_body(x_ref, w_ref, b_ref, emax_ref, t0_ref, h_ref, m_ref):
    h = jnp.dot(x_ref[...], w_ref[...], preferred_element_type=_F32)
    h = h + b_ref[...]
    h_ref[...] = h
    hmax = jnp.max(h, axis=0, keepdims=True)
    m_ref[...] = t0_ref[...] * jnp.maximum(hmax + emax_ref[...], 0.0)


def _node_encode(x, enc_W, enc_b, eemax, t0_row):
    return pl.pallas_call(
        _node_enc_body,
        out_shape=[
            jax.ShapeDtypeStruct((_N, _EMB), _F32),
            jax.ShapeDtypeStruct((1, _EMB), _F32),
        ],
    )(x, enc_W, enc_b.reshape(1, _EMB), eemax, t0_row)


def _aggr_from(o, hin):
    pm = o[:, :_HALF]
    den = o[:, _HALF:]
    return jnp.where(den > 0.0, pm / den + _EPS, 0.0) + hin


def _bn_tc(y, g, b):
    mu = jnp.mean(y, axis=0, keepdims=True)
    yc = y - mu
    var = jnp.mean(yc * yc, axis=0, keepdims=True)
    return yc * (g / jnp.sqrt(var + 1e-5)) + b


def _layer_body(has_resid, is_final, *refs):
    if has_resid:
        (olo, ohi, hin, hprev, w1, b1, g1, be1, w2, b2,
         ng, nb, emax, tnext, *outs) = refs
    else:
        (olo, ohi, hin, w1, b1, g1, be1, w2, b2,
         ng, nb, emax, tnext, *outs) = refs
        hprev = None

    hin_v = hin[...]
    out_lo = _aggr_from(olo[...][:_N], hin_v[:, :_HALF])
    out_hi = _aggr_from(ohi[...][:_N], hin_v[:, _HALF:])
    out = jnp.concatenate([out_lo, out_hi], axis=1)
    y = jnp.dot(out, w1[...], preferred_element_type=_F32) + b1[...]
    y = _bn_tc(y, g1[...], be1[...])
    y = jnp.maximum(y, 0.0)
    hnew = jnp.dot(y, w2[...], preferred_element_type=_F32) + b2[...]
    if hprev is not None:
        hnew = hnew + hprev[...]

    hb = _bn_tc(hnew, ng[...], nb[...])
    hin2 = jnp.maximum(hb, 0.0)
    if is_final:
        (hf_ref,) = outs
        hf_ref[...] = hin2
    else:
        (hnew_ref, hin2_ref, m_ref) = outs
        hnew_ref[...] = hnew
        hin2_ref[...] = hin2
        hmax = jnp.max(hin2, axis=0, keepdims=True)
        m_ref[...] = tnext[...] * jnp.maximum(hmax + emax[...], 0.0)


def _layer_tc(olo, ohi, hin, hprev, w1, b1, g1, be1, w2, b2,
              ng, nb, eemax, tnext_row, is_final):
    has_resid = hprev is not None
    if is_final:
        out_shape = [jax.ShapeDtypeStruct((_N, _EMB), _F32)]
    else:
        out_shape = [
            jax.ShapeDtypeStruct((_N, _EMB), _F32),
            jax.ShapeDtypeStruct((_N, _EMB), _F32),
            jax.ShapeDtypeStruct((1, _EMB), _F32),
        ]
    args = [olo, ohi, hin]
    if has_resid:
        args.append(hprev)
    args += [w1, b1.reshape(1, -1), g1.reshape(1, -1), be1.reshape(1, -1),
             w2, b2.reshape(1, -1), ng.reshape(1, -1), nb.reshape(1, -1),
             eemax, tnext_row]
    body = functools.partial(_layer_body, has_resid, is_final)
    return pl.pallas_call(body, out_shape=out_shape)(*args)


def _pool_body(hf_ref, batch_ref, pw_ref, pb_ref, rf_ref, beta_ref, out_ref):
    onehot = (lax.broadcasted_iota(jnp.int32, (_G, _N), 0)
              == batch_ref[...]).astype(_F32)
    sums = jnp.dot(onehot, hf_ref[...], preferred_element_type=_F32)
    counts = jnp.sum(onehot, axis=1, keepdims=True)
    hg = sums / jnp.maximum(counts, 1.0)
    pred = jnp.dot(hg, pw_ref[...], preferred_element_type=_F32) + pb_ref[...]
    sig = 1.0 / (1.0 + jnp.exp(-pred))
    out_ref[...] = (1.0 - beta_ref[...]) * sig + beta_ref[...] * rf_ref[...]


def _pool(hf, batch, pred_W, pred_b, rf_pred, beta):
    pw = jnp.zeros((_EMB, _EMB), _F32).at[:, 0:1].set(pred_W)
    pb = jnp.zeros((1, _EMB), _F32).at[0, 0].set(pred_b[0])
    rf = jnp.broadcast_to(rf_pred.reshape(_G, 1), (_G, _EMB))
    beta_row = jnp.broadcast_to(beta.reshape(1, 1), (1, _EMB))
    res = pl.pallas_call(
        _pool_body,
        out_shape=jax.ShapeDtypeStruct((_G, _EMB), _F32),
    )(hf, batch.reshape(1, _N), pw, pb, rf, beta_row)
    return res[:, 0:1]


def kernel(x, edge_index, edge_attr, batch, rf_pred, enc_W, enc_b, edge_W,
           edge_b, W1, b1, g1, be1, W2, b2, t_param, norm_g, norm_b, pred_W,
           pred_b, beta):
    eidx4 = edge_index.reshape(2, _NS, _NCHUNK, _C)

    eelo, eehi, eemax = _edge_encode(edge_attr, edge_W, edge_b)

    t_rows = [jnp.broadcast_to(t_param[l].reshape(1, 1), (1, _EMB))
              for l in range(_L)]
    t_vecs = [jnp.broadcast_to(t_param[l].reshape(1), (16,)) for l in range(_L)]

    hin, m_cur = _node_encode(x, enc_W, enc_b, eemax, t_rows[0])

    hprev = None
    hf = None
    for l in range(_L):
        olo, ohi = _sc_aggregate(hin, eelo, eehi, eidx4,
                                 m_cur[0, :_HALF], m_cur[0, _HALF:],
                                 t_vecs[l])
        is_final = l == _L - 1
        tnext = t_rows[l + 1] if not is_final else t_rows[l]
        outs = _layer_tc(olo, ohi, hin, hprev, W1[l], b1[l], g1[l],
                         be1[l], W2[l], b2[l], norm_g[l], norm_b[l], eemax,
                         tnext, is_final)
        if is_final:
            (hf,) = outs
        else:
            hprev, hin, m_cur = outs

    return _pool(hf, batch, pred_W, pred_b, rf_pred, beta)


# speedup vs baseline: 2.4943x; 1.0170x over previous
import functools

import jax
import jax.numpy as jnp
from jax import lax
from jax.experimental import pallas as pl
from jax.experimental.pallas import tpu as pltpu
from jax.experimental.pallas import tpu_sc as plsc

_N = 10000
_E = 160000
_EMB = 128
_HALF = 64
_G = 128
_L = 4
_EPS = 1e-7

_NC = 2
_NS = 16
_C = 40
_EPT = _E // _NS
_NCHUNK = _EPT // _C
_NP = 10240
_RPT = _NP // _NS
_ZR = _C

_F32 = jnp.float32


def _sc_body(hfull, eelo, eehi, eidx4, mlo, mhi, tvec, olo, ohi,
             acc, gb0, gb1, eb0, eb1, pp0, pp1, ib0, ib1, db0, db1, mv, tb,
             sg0, sg1, se0, se1, ss0, ss1, si0, si1, sd0, sd1, sw0, sw1):
    c = lax.axis_index("c")
    s = lax.axis_index("s")
    gb = (gb0, gb1)
    eb = (eb0, eb1)
    pp = (pp0, pp1)
    ib = (ib0, ib1)
    db = (db0, db1)
    si = (si0, si1)
    sd = (sd0, sd1)
    zb = (pp0, pp1)
    sg = (sg0, sg1)
    se = (se0, se1)
    ss = (ss0, ss1)
    sw = (sw0, sw1)

    @pl.when(c == 0)
    def _():
        pltpu.sync_copy(mlo, mv)

    @pl.when(c == 1)
    def _():
        pltpu.sync_copy(mhi, mv)

    pltpu.sync_copy(tvec, tb)
    tv = tb[...]
    mj = [mv[pl.ds(16 * j, 16)] for j in range(4)]

    @pl.loop(0, _ZR)
    def _zbl(i):
        for j in range(8):
            pp0[i, pl.ds(16 * j, 16)] = jnp.zeros((16,), _F32)

    row0 = s * _RPT
    nzc = _RPT // _ZR

    for z in range(nzc):
        rows = pl.ds(pl.multiple_of(row0 + z * _ZR, 8), _ZR)
        pltpu.async_copy(pp0, acc.at[rows], sw0)
    for z in range(nzc):
        rows = pl.ds(pl.multiple_of(row0 + z * _ZR, 8), _ZR)
        pltpu.make_async_copy(pp0, acc.at[rows], sw0).wait()

    plsc.subcore_barrier()

    ebase = s * _EPT

    def _issue_src(k, sl):
        pltpu.async_copy(eidx4.at[0, s, k], ib[sl], si[sl])

    def _wait_src(sl):
        pltpu.make_async_copy(eidx4.at[0, s, 0], ib[sl], si[sl]).wait()

    def _issue_dst(k, sl):
        pltpu.async_copy(eidx4.at[1, s, k], db[sl], sd[sl])

    def _wait_dst(sl):
        pltpu.make_async_copy(eidx4.at[1, s, 0], db[sl], sd[sl]).wait()

    def _issue(k, sl):
        base = pl.multiple_of(ebase + k * _C, 8)
        pltpu.async_copy(hfull.at[ib[sl]], gb[sl], sg[sl])

        @pl.when(c == 0)
        def _():
            pltpu.async_copy(eelo.at[pl.ds(base, _C)], eb[sl], se[sl])

        @pl.when(c == 1)
        def _():
            pltpu.async_copy(eehi.at[pl.ds(base, _C)], eb[sl], se[sl])

    def _wait_in(sl):
        pltpu.make_async_copy(hfull.at[ib[sl]], gb[sl], sg[sl]).wait()
        pltpu.make_async_copy(eelo.at[pl.ds(0, _C)], eb[sl], se[sl]).wait()

    def _issue_sc(sl):
        pltpu.async_copy(pp[sl], acc.at[db[sl]], ss[sl], add=True)

    def _wait_sc(sl):
        pltpu.make_async_copy(pp[sl], acc.at[db[sl]], ss[sl]).wait()

    def _compute(k, sl):
        gbuf = gb[sl]
        eebuf = eb[sl]
        pmp = pp[sl]

        def _body(goff):
            @plsc.parallel_loop(0, _C, unroll=20)
            def _row(r):
                for j in range(4):
                    g = gbuf[r, pl.ds(goff + 16 * j, 16)]
                    e = eebuf[r, pl.ds(16 * j, 16)]
                    rr = jnp.maximum(g + e, 0.0)
                    p = jnp.exp(rr * tv - mj[j])
                    pmp[r, pl.ds(16 * j, 16)] = p * rr
                    pmp[r, pl.ds(_HALF + 16 * j, 16)] = p

        @pl.when(c == 0)
        def _():
            _body(0)

        @pl.when(c == 1)
        def _():
            _body(_HALF)

    _issue_src(0, 0)
    _issue_src(1, 1)
    _wait_src(0)
    _issue(0, 0)
    _wait_src(1)
    _issue(1, 1)

    npair = _NCHUNK // 2

    @pl.loop(0, npair)
    def _pair(m):
        a = 2 * m
        for par in range(2):
            k = a + par
            _wait_in(par)

            @pl.when(m > 0)
            def _():
                _wait_sc(par)

            _issue_dst(k, par)

            @pl.when(k + 2 < _NCHUNK)
            def _():
                _issue_src(k + 2, par)

            _compute(k, par)
            _wait_dst(par)
            _issue_sc(par)

            @pl.when(k + 2 < _NCHUNK)
            def _():
                _wait_src(par)
                _issue(k + 2, par)

    _wait_sc(0)
    _wait_sc(1)

    plsc.subcore_barrier()

    def _wo_one(z, par):
        rows = pl.ds(pl.multiple_of(row0 + z * _ZR, 8), _ZR)
        pltpu.sync_copy(acc.at[rows], zb[par])

        @pl.when(c == 0)
        def _():
            pltpu.async_copy(zb[par], olo.at[rows], sw[par])

        @pl.when(c == 1)
        def _():
            pltpu.async_copy(zb[par], ohi.at[rows], sw[par])

    def _wo_wait(par):
        pltpu.make_async_copy(zb[par], olo.at[pl.ds(0, _ZR)], sw[par]).wait()

    @pl.loop(0, nzc // 2)
    def _wo(z):
        for par in range(2):
            @pl.when(z > 0)
            def _():
                _wo_wait(par)

            _wo_one(2 * z + par, par)

    _wo_wait(0)
    _wo_wait(1)


_SC_KERNEL_CACHE = []


def _sc_aggregate(*args):
    if not _SC_KERNEL_CACHE:
        _SC_KERNEL_CACHE.append(_build_sc_kernel())
    return _SC_KERNEL_CACHE[0](*args)


def _build_sc_kernel():
    return pl.kernel(
        _sc_body,
        out_type=(
            jax.ShapeDtypeStruct((_NP, 2 * _HALF), _F32),
            jax.ShapeDtypeStruct((_NP, 2 * _HALF), _F32),
        ),
        mesh=plsc.VectorSubcoreMesh(
            core_axis_name="c", subcore_axis_name="s",
            num_cores=_NC, num_subcores=_NS),
        scratch_types=[
        pltpu.VMEM_SHARED((_NP, 2 * _HALF), _F32),
        pltpu.VMEM((_C, 2 * _HALF), _F32),
        pltpu.VMEM((_C, 2 * _HALF), _F32),
        pltpu.VMEM((_C, _HALF), _F32),
        pltpu.VMEM((_C, _HALF), _F32),
        pltpu.VMEM((_C, 2 * _HALF), _F32),
        pltpu.VMEM((_C, 2 * _HALF), _F32),
        pltpu.VMEM((_C,), jnp.int32),
        pltpu.VMEM((_C,), jnp.int32),
        pltpu.VMEM((_C,), jnp.int32),
        pltpu.VMEM((_C,), jnp.int32),
        pltpu.VMEM((_HALF,), _F32),
        pltpu.VMEM((16,), _F32),
        pltpu.SemaphoreType.DMA,
        pltpu.SemaphoreType.DMA,
        pltpu.SemaphoreType.DMA,
        pltpu.SemaphoreType.DMA,
        pltpu.SemaphoreType.DMA,
        pltpu.SemaphoreType.DMA,
        pltpu.SemaphoreType.DMA,
        pltpu.SemaphoreType.DMA,
        pltpu.SemaphoreType.DMA,
        pltpu.SemaphoreType.DMA,
        pltpu.SemaphoreType.DMA,
        pltpu.SemaphoreType.DMA,
        ],
    )


def _edge_enc_body(ea_ref, w_ref, b_ref, eelo_ref, eehi_ref, emax_ref, accm):
    i = pl.program_id(0)
    ee = jnp.dot(ea_ref[...], w_ref[...], preferred_element_type=_F32)
    ee = ee + b_ref[...]
    eelo_ref[...] = ee[:, :_HALF]
    eehi_ref[...] = ee[:, _HALF:]
    bm = jnp.max(ee, axis=0, keepdims=True)

    @pl.when(i == 0)
    def _():
        accm[...] = bm

    @pl.when(i > 0)
    def _():
        accm[...] = jnp.maximum(accm[...], bm)

    emax_ref[...] = accm[...]


_TE = 4000


def _edge_encode(edge_attr, edge_W, edge_b):
    nblk = _E // _TE
    return pl.pallas_call(
        _edge_enc_body,
        grid=(nblk,),
        in_specs=[
            pl.BlockSpec((_TE, 16), lambda i: (i, 0)),
            pl.BlockSpec((16, _EMB), lambda i: (0, 0)),
            pl.BlockSpec((1, _EMB), lambda i: (0, 0)),
        ],
        out_specs=[
            pl.BlockSpec((_TE, _HALF), lambda i: (i, 0)),
            pl.BlockSpec((_TE, _HALF), lambda i: (i, 0)),
            pl.BlockSpec((1, _EMB), lambda i: (0, 0)),
        ],
        out_shape=[
            jax.ShapeDtypeStruct((_E, _HALF), _F32),
            jax.ShapeDtypeStruct((_E, _HALF), _F32),
            jax.ShapeDtypeStruct((1, _EMB), _F32),
        ],
        scratch_shapes=[pltpu.VMEM((1, _EMB), _F32)],
    )(edge_attr, edge_W, edge_b.reshape(1, _EMB))


def _node_enc_body(x_ref, w_ref, b_ref, emax_ref, t0_ref, h_ref, m_ref):
    h = jnp.dot(x_ref[...], w_ref[...], preferred_element_type=_F32)
    h = h + b_ref[...]
    h_ref[...] = h
    hmax = jnp.max(h, axis=0, keepdims=True)
    m_ref[...] = t0_ref[...] * jnp.maximum(hmax + emax_ref[...], 0.0)


def _node_encode(x, enc_W, enc_b, eemax, t0_row):
    return pl.pallas_call(
        _node_enc_body,
        out_shape=[
            jax.ShapeDtypeStruct((_N, _EMB), _F32),
            jax.ShapeDtypeStruct((1, _EMB), _F32),
        ],
    )(x, enc_W, enc_b.reshape(1, _EMB), eemax, t0_row)


def _aggr_from(o, hin):
    pm = o[:, :_HALF]
    den = o[:, _HALF:]
    return jnp.where(den > 0.0, pm / den + _EPS, 0.0) + hin


def _bn_tc(y, g, b):
    mu = jnp.mean(y, axis=0, keepdims=True)
    yc = y - mu
    var = jnp.mean(yc * yc, axis=0, keepdims=True)
    return yc * (g / jnp.sqrt(var + 1e-5)) + b


def _layer_body(has_resid, is_final, *refs):
    if has_resid:
        (olo, ohi, hin, hprev, w1, b1, g1, be1, w2, b2,
         ng, nb, emax, tnext, *outs) = refs
    else:
        (olo, ohi, hin, w1, b1, g1, be1, w2, b2,
         ng, nb, emax, tnext, *outs) = refs
        hprev = None

    hin_v = hin[...]
    out_lo = _aggr_from(olo[...][:_N], hin_v[:, :_HALF])
    out_hi = _aggr_from(ohi[...][:_N], hin_v[:, _HALF:])
    out = jnp.concatenate([out_lo, out_hi], axis=1)
    y = jnp.dot(out, w1[...], preferred_element_type=_F32) + b1[...]
    y = _bn_tc(y, g1[...], be1[...])
    y = jnp.maximum(y, 0.0)
    hnew = jnp.dot(y, w2[...], preferred_element_type=_F32) + b2[...]
    if hprev is not None:
        hnew = hnew + hprev[...]

    hb = _bn_tc(hnew, ng[...], nb[...])
    hin2 = jnp.maximum(hb, 0.0)
    if is_final:
        (hf_ref,) = outs
        hf_ref[...] = hin2
    else:
        (hnew_ref, hin2_ref, m_ref) = outs
        hnew_ref[...] = hnew
        hin2_ref[...] = hin2
        hmax = jnp.max(hin2, axis=0, keepdims=True)
        m_ref[...] = tnext[...] * jnp.maximum(hmax + emax[...], 0.0)


def _layer_tc(olo, ohi, hin, hprev, w1, b1, g1, be1, w2, b2,
              ng, nb, eemax, tnext_row, is_final):
    has_resid = hprev is not None
    if is_final:
        out_shape = [jax.ShapeDtypeStruct((_N, _EMB), _F32)]
    else:
        out_shape = [
            jax.ShapeDtypeStruct((_N, _EMB), _F32),
            jax.ShapeDtypeStruct((_N, _EMB), _F32),
            jax.ShapeDtypeStruct((1, _EMB), _F32),
        ]
    args = [olo, ohi, hin]
    if has_resid:
        args.append(hprev)
    args += [w1, b1.reshape(1, -1), g1.reshape(1, -1), be1.reshape(1, -1),
             w2, b2.reshape(1, -1), ng.reshape(1, -1), nb.reshape(1, -1),
             eemax, tnext_row]
    body = functools.partial(_layer_body, has_resid, is_final)
    return pl.pallas_call(body, out_shape=out_shape)(*args)


def _pool_body(hf_ref, batch_ref, pw_ref, pb_ref, rf_ref, beta_ref, out_ref):
    onehot = (lax.broadcasted_iota(jnp.int32, (_G, _N), 0)
              == batch_ref[...]).astype(_F32)
    sums = jnp.dot(onehot, hf_ref[...], preferred_element_type=_F32)
    counts = jnp.sum(onehot, axis=1, keepdims=True)
    hg = sums / jnp.maximum(counts, 1.0)
    pred = jnp.dot(hg, pw_ref[...], preferred_element_type=_F32) + pb_ref[...]
    sig = 1.0 / (1.0 + jnp.exp(-pred))
    out_ref[...] = (1.0 - beta_ref[...]) * sig + beta_ref[...] * rf_ref[...]


def _pool(hf, batch, pred_W, pred_b, rf_pred, beta):
    pw = jnp.zeros((_EMB, _EMB), _F32).at[:, 0:1].set(pred_W)
    pb = jnp.zeros((1, _EMB), _F32).at[0, 0].set(pred_b[0])
    rf = jnp.broadcast_to(rf_pred.reshape(_G, 1), (_G, _EMB))
    beta_row = jnp.broadcast_to(beta.reshape(1, 1), (1, _EMB))
    res = pl.pallas_call(
        _pool_body,
        out_shape=jax.ShapeDtypeStruct((_G, _EMB), _F32),
    )(hf, batch.reshape(1, _N), pw, pb, rf, beta_row)
    return res[:, 0:1]


def kernel(x, edge_index, edge_attr, batch, rf_pred, enc_W, enc_b, edge_W,
           edge_b, W1, b1, g1, be1, W2, b2, t_param, norm_g, norm_b, pred_W,
           pred_b, beta):
    eidx4 = edge_index.reshape(2, _NS, _NCHUNK, _C)

    eelo, eehi, eemax = _edge_encode(edge_attr, edge_W, edge_b)

    t_rows = [jnp.broadcast_to(t_param[l].reshape(1, 1), (1, _EMB))
              for l in range(_L)]
    t_vecs = [jnp.broadcast_to(t_param[l].reshape(1), (16,)) for l in range(_L)]

    hin, m_cur = _node_encode(x, enc_W, enc_b, eemax, t_rows[0])

    hprev = None
    hf = None
    for l in range(_L):
        olo, ohi = _sc_aggregate(hin, eelo, eehi, eidx4,
                                 m_cur[0, :_HALF], m_cur[0, _HALF:],
                                 t_vecs[l])
        is_final = l == _L - 1
        tnext = t_rows[l + 1] if not is_final else t_rows[l]
        outs = _layer_tc(olo, ohi, hin, hprev, W1[l], b1[l], g1[l],
                         be1[l], W2[l], b2[l], norm_g[l], norm_b[l], eemax,
                         tnext, is_final)
        if is_final:
            (hf,) = outs
        else:
            hprev, hin, m_cur = outs

    return _pool(hf, batch, pred_W, pred_b, rf_pred, beta)
